# Initial kernel scaffold; baseline (speedup 1.0000x reference)
import jax
import jax.numpy as jnp
from jax.experimental import pallas as pl


def kernel(x, edge_index, edge_attr, W1, b1, W2, b2):
    raise NotImplementedError("write your pallas kernel here")


# speedup vs baseline: 5.2424x; 5.2424x over previous
import jax
import jax.numpy as jnp
from jax import lax
from jax.experimental import pallas as pl
from jax.experimental.pallas import tpu as pltpu
from jax.experimental.pallas import tpu_sc as plsc

N_NODES = 10000
N_EDGES = 320000
H = 128

NC = 2
NS = 16
EPW = N_EDGES // (NC * NS)
CH = 128
NFULL = EPW // CH
TAIL = EPW - NFULL * CH
N_PAD = 10240
RPT = N_PAD // NS
ZR = 128
NZ = RPT // ZR


def _sc_body(z_hbm, row_hbm, col_hbm, ea_hbm, out_hbm,
             acc, ebuf, tebuf, ribuf, cibuf, tribuf, tcibuf, zbuf):
    cid = lax.axis_index("c")
    sid = lax.axis_index("s")
    base = (cid * NS + sid) * EPW

    pltpu.sync_copy(z_hbm, zbuf)
    for z in range(NZ):
        pltpu.sync_copy(zbuf, acc.at[pl.ds(sid * RPT + z * ZR, ZR)])
    plsc.subcore_barrier()

    def chunk(k, carry):
        e0 = base + k * CH
        pltpu.sync_copy(ea_hbm.at[pl.ds(e0, CH)], ebuf)
        pltpu.sync_copy(row_hbm.at[pl.ds(e0, CH)], ribuf)
        pltpu.sync_copy(col_hbm.at[pl.ds(e0, CH)], cibuf)
        pltpu.sync_copy(ebuf, acc.at[ribuf], add=True)
        pltpu.sync_copy(ebuf, acc.at[cibuf], add=True)
        return carry

    lax.fori_loop(0, NFULL, chunk, 0)

    e0 = base + NFULL * CH
    pltpu.sync_copy(ea_hbm.at[pl.ds(e0, TAIL)], tebuf)
    pltpu.sync_copy(row_hbm.at[pl.ds(e0, TAIL)], tribuf)
    pltpu.sync_copy(col_hbm.at[pl.ds(e0, TAIL)], tcibuf)
    pltpu.sync_copy(tebuf, acc.at[tribuf], add=True)
    pltpu.sync_copy(tebuf, acc.at[tcibuf], add=True)

    plsc.subcore_barrier()

    for z in range(NZ):
        r0 = sid * RPT + z * ZR
        pltpu.sync_copy(acc.at[pl.ds(r0, ZR)], zbuf)
        pltpu.sync_copy(zbuf, out_hbm.at[cid, pl.ds(r0, ZR)])


_sc_scatter = pl.kernel(
    _sc_body,
    out_type=jax.ShapeDtypeStruct((NC, N_PAD, H), jnp.float32),
    mesh=plsc.VectorSubcoreMesh(core_axis_name="c", subcore_axis_name="s"),
    scratch_types=[
        pltpu.VMEM_SHARED((N_PAD, H), jnp.float32),
        pltpu.VMEM((CH, H), jnp.float32),
        pltpu.VMEM((TAIL, H), jnp.float32),
        pltpu.VMEM((CH,), jnp.int32),
        pltpu.VMEM((CH,), jnp.int32),
        pltpu.VMEM((TAIL,), jnp.int32),
        pltpu.VMEM((TAIL,), jnp.int32),
        pltpu.VMEM((ZR, H), jnp.float32),
    ],
)


BR = 1000


def _mlp_body(x_ref, p0_ref, p1_ref, w1xt_ref, w1at_ref, b1_ref,
              w2t_ref, b2_ref, o_ref):
    aggr = p0_ref[...] + p1_ref[...]
    h = jnp.dot(x_ref[...], w1xt_ref[...], preferred_element_type=jnp.float32)
    h = h + jnp.dot(aggr, w1at_ref[...], preferred_element_type=jnp.float32)
    h = jnp.maximum(h + b1_ref[...], 0.0)
    o_ref[...] = (
        jnp.dot(h, w2t_ref[...], preferred_element_type=jnp.float32)
        + b2_ref[...]
    )


_mlp = pl.pallas_call(
    _mlp_body,
    grid=(N_NODES // BR,),
    in_specs=[
        pl.BlockSpec((BR, H), lambda i: (i, 0)),
        pl.BlockSpec((BR, H), lambda i: (i, 0)),
        pl.BlockSpec((BR, H), lambda i: (i, 0)),
        pl.BlockSpec((H, H), lambda i: (0, 0)),
        pl.BlockSpec((H, H), lambda i: (0, 0)),
        pl.BlockSpec((1, H), lambda i: (0, 0)),
        pl.BlockSpec((H, H), lambda i: (0, 0)),
        pl.BlockSpec((1, H), lambda i: (0, 0)),
    ],
    out_specs=pl.BlockSpec((BR, H), lambda i: (i, 0)),
    out_shape=jax.ShapeDtypeStruct((N_NODES, H), jnp.float32),
)


def kernel(x, edge_index, edge_attr, W1, b1, W2, b2):
    row = edge_index[0].astype(jnp.int32)
    col = edge_index[1].astype(jnp.int32)
    zeros = jnp.zeros((ZR, H), jnp.float32)
    partials = _sc_scatter(zeros, row, col, edge_attr)[:, :N_NODES, :]
    return _mlp(x, partials[0], partials[1],
                W1[:, :H].T, W1[:, H:].T, b1[None, :],
                W2.T, b2[None, :])


# speedup vs baseline: 8.8163x; 1.6817x over previous
import jax
import jax.numpy as jnp
from jax import lax
from jax.experimental import pallas as pl
from jax.experimental.pallas import tpu as pltpu
from jax.experimental.pallas import tpu_sc as plsc

N_NODES = 10000
N_EDGES = 320000
H = 128

NC = 2
NS = 16
EPW = N_EDGES // (NC * NS)
CH = 128
NFULL = EPW // CH
TAIL = EPW - NFULL * CH
N_PAD = 10240
RPT = N_PAD // NS
ZR = 128
NZ = RPT // ZR


NBUF = 2
NGRP = NFULL // NBUF


def _sc_body(z_hbm, row_hbm, col_hbm, ea_hbm, out_hbm,
             acc, ebuf, tebuf, ribuf, cibuf, tribuf, tcibuf,
             csem, ssem):
    cid = lax.axis_index("c")
    sid = lax.axis_index("s")
    base = (cid * NS + sid) * EPW

    def start_in(k, b):
        e0 = base + k * CH
        pltpu.async_copy(ea_hbm.at[pl.ds(e0, CH)], ebuf.at[b], csem.at[b])
        pltpu.async_copy(row_hbm.at[pl.ds(e0, CH)], ribuf.at[b], csem.at[b])
        pltpu.async_copy(col_hbm.at[pl.ds(e0, CH)], cibuf.at[b], csem.at[b])

    def wait_in(b):
        pltpu.make_async_copy(ea_hbm.at[pl.ds(0, CH)], ebuf.at[b],
                              csem.at[b]).wait()
        pltpu.make_async_copy(row_hbm.at[pl.ds(0, CH)], ribuf.at[b],
                              csem.at[b]).wait()
        pltpu.make_async_copy(col_hbm.at[pl.ds(0, CH)], cibuf.at[b],
                              csem.at[b]).wait()

    def wait_scat(b):
        pltpu.make_async_copy(ebuf.at[b], acc.at[ribuf.at[b]],
                              ssem.at[b]).wait()
        pltpu.make_async_copy(ebuf.at[b], acc.at[cibuf.at[b]],
                              ssem.at[b]).wait()

    pltpu.sync_copy(z_hbm, ebuf.at[0])
    for z in range(NZ):
        pltpu.sync_copy(ebuf.at[0], acc.at[pl.ds(sid * RPT + z * ZR, ZR)])
    plsc.subcore_barrier()

    start_in(0, 0)

    def group(g, carry):
        for b in range(NBUF):
            k = g * NBUF + b
            bn = (b + 1) % NBUF
            wait_in(b)
            pltpu.async_copy(ebuf.at[b], acc.at[ribuf.at[b]], ssem.at[b],
                             add=True)
            pltpu.async_copy(ebuf.at[b], acc.at[cibuf.at[b]], ssem.at[b],
                             add=True)

            @pl.when(k >= NBUF - 1)
            def _():
                wait_scat(bn)

            @pl.when(k + 1 < NFULL)
            def _():
                start_in(k + 1, bn)
        return carry

    lax.fori_loop(0, NGRP, group, 0)
    wait_scat((NFULL - 1) % NBUF)

    e0 = base + NFULL * CH
    pltpu.sync_copy(ea_hbm.at[pl.ds(e0, TAIL)], tebuf)
    pltpu.sync_copy(row_hbm.at[pl.ds(e0, TAIL)], tribuf)
    pltpu.sync_copy(col_hbm.at[pl.ds(e0, TAIL)], tcibuf)
    pltpu.sync_copy(tebuf, acc.at[tribuf], add=True)
    pltpu.sync_copy(tebuf, acc.at[tcibuf], add=True)

    plsc.subcore_barrier()

    for z in range(NZ):
        r0 = sid * RPT + z * ZR
        pltpu.sync_copy(acc.at[pl.ds(r0, ZR)], ebuf.at[0])
        pltpu.sync_copy(ebuf.at[0], out_hbm.at[cid, pl.ds(r0, ZR)])


_sc_scatter = pl.kernel(
    _sc_body,
    out_type=jax.ShapeDtypeStruct((NC, N_PAD, H), jnp.float32),
    mesh=plsc.VectorSubcoreMesh(core_axis_name="c", subcore_axis_name="s"),
    scratch_types=[
        pltpu.VMEM_SHARED((N_PAD, H), jnp.float32),
        pltpu.VMEM((NBUF, CH, H), jnp.float32),
        pltpu.VMEM((TAIL, H), jnp.float32),
        pltpu.VMEM((NBUF, CH), jnp.int32),
        pltpu.VMEM((NBUF, CH), jnp.int32),
        pltpu.VMEM((TAIL,), jnp.int32),
        pltpu.VMEM((TAIL,), jnp.int32),
        pltpu.SemaphoreType.DMA((NBUF,)),
        pltpu.SemaphoreType.DMA((NBUF,)),
    ],
)


BR = 1000


def _mlp_body(x_ref, p0_ref, p1_ref, w1xt_ref, w1at_ref, b1_ref,
              w2t_ref, b2_ref, o_ref):
    aggr = p0_ref[...] + p1_ref[...]
    h = jnp.dot(x_ref[...], w1xt_ref[...], preferred_element_type=jnp.float32)
    h = h + jnp.dot(aggr, w1at_ref[...], preferred_element_type=jnp.float32)
    h = jnp.maximum(h + b1_ref[...], 0.0)
    o_ref[...] = (
        jnp.dot(h, w2t_ref[...], preferred_element_type=jnp.float32)
        + b2_ref[...]
    )


_mlp = pl.pallas_call(
    _mlp_body,
    grid=(N_NODES // BR,),
    in_specs=[
        pl.BlockSpec((BR, H), lambda i: (i, 0)),
        pl.BlockSpec((BR, H), lambda i: (i, 0)),
        pl.BlockSpec((BR, H), lambda i: (i, 0)),
        pl.BlockSpec((H, H), lambda i: (0, 0)),
        pl.BlockSpec((H, H), lambda i: (0, 0)),
        pl.BlockSpec((1, H), lambda i: (0, 0)),
        pl.BlockSpec((H, H), lambda i: (0, 0)),
        pl.BlockSpec((1, H), lambda i: (0, 0)),
    ],
    out_specs=pl.BlockSpec((BR, H), lambda i: (i, 0)),
    out_shape=jax.ShapeDtypeStruct((N_NODES, H), jnp.float32),
)


def kernel(x, edge_index, edge_attr, W1, b1, W2, b2):
    row = edge_index[0].astype(jnp.int32)
    col = edge_index[1].astype(jnp.int32)
    zeros = jnp.zeros((ZR, H), jnp.float32)
    partials = _sc_scatter(zeros, row, col, edge_attr)[:, :N_NODES, :]
    return _mlp(x, partials[0], partials[1],
                W1[:, :H].T, W1[:, H:].T, b1[None, :],
                W2.T, b2[None, :])


# speedup vs baseline: 9.2209x; 1.0459x over previous
import jax
import jax.numpy as jnp
from jax import lax
from jax.experimental import pallas as pl
from jax.experimental.pallas import tpu as pltpu
from jax.experimental.pallas import tpu_sc as plsc

N_NODES = 10000
N_EDGES = 320000
H = 128

NC = 2
NS = 16
EPW = N_EDGES // (NC * NS)
CH = 128
NFULL = EPW // CH
TAIL = EPW - NFULL * CH
N_PAD = 10240
RPT = N_PAD // NS
ZR = 128
NZ = RPT // ZR


NBUF = 2
NGRP = NFULL // NBUF


def _sc_body(z_hbm, row_hbm, col_hbm, ea_hbm, out_hbm,
             acc, ebuf, tebuf, ribuf, cibuf, tribuf, tcibuf,
             csem, ssem):
    cid = lax.axis_index("c")
    sid = lax.axis_index("s")
    base = (cid * NS + sid) * EPW

    def start_in(k, b):
        e0 = base + k * CH
        pltpu.async_copy(ea_hbm.at[pl.ds(e0, CH)], ebuf.at[b], csem.at[b])
        pltpu.async_copy(row_hbm.at[pl.ds(e0, CH)], ribuf.at[b], csem.at[b])
        pltpu.async_copy(col_hbm.at[pl.ds(e0, CH)], cibuf.at[b], csem.at[b])

    def wait_in(b):
        pltpu.make_async_copy(ea_hbm.at[pl.ds(0, CH)], ebuf.at[b],
                              csem.at[b]).wait()
        pltpu.make_async_copy(row_hbm.at[pl.ds(0, CH)], ribuf.at[b],
                              csem.at[b]).wait()
        pltpu.make_async_copy(col_hbm.at[pl.ds(0, CH)], cibuf.at[b],
                              csem.at[b]).wait()

    def wait_scat(b):
        pltpu.make_async_copy(ebuf.at[b], acc.at[ribuf.at[b]],
                              ssem.at[b]).wait()
        pltpu.make_async_copy(ebuf.at[b], acc.at[cibuf.at[b]],
                              ssem.at[b]).wait()

    pltpu.sync_copy(z_hbm, ebuf.at[0])
    for z in range(NZ):
        pltpu.sync_copy(ebuf.at[0], acc.at[pl.ds(sid * RPT + z * ZR, ZR)])
    plsc.subcore_barrier()

    start_in(0, 0)

    def group(g, carry):
        for b in range(NBUF):
            k = g * NBUF + b
            bn = (b + 1) % NBUF
            wait_in(b)
            pltpu.async_copy(ebuf.at[b], acc.at[ribuf.at[b]], ssem.at[b],
                             add=True)
            pltpu.async_copy(ebuf.at[b], acc.at[cibuf.at[b]], ssem.at[b],
                             add=True)

            @pl.when(k >= NBUF - 1)
            def _():
                wait_scat(bn)

            @pl.when(k + 1 < NFULL)
            def _():
                start_in(k + 1, bn)
        return carry

    lax.fori_loop(0, NGRP, group, 0)
    wait_scat((NFULL - 1) % NBUF)

    e0 = base + NFULL * CH
    pltpu.sync_copy(ea_hbm.at[pl.ds(e0, TAIL)], tebuf)
    pltpu.sync_copy(row_hbm.at[pl.ds(e0, TAIL)], tribuf)
    pltpu.sync_copy(col_hbm.at[pl.ds(e0, TAIL)], tcibuf)
    pltpu.sync_copy(tebuf, acc.at[tribuf], add=True)
    pltpu.sync_copy(tebuf, acc.at[tcibuf], add=True)

    plsc.subcore_barrier()

    for z in range(NZ):
        s = z % NBUF
        r0 = sid * RPT + z * ZR
        if z >= NBUF:
            rp = sid * RPT + (z - NBUF) * ZR
            pltpu.make_async_copy(
                ebuf.at[s], out_hbm.at[cid, pl.ds(rp, ZR)], csem.at[s]).wait()
        pltpu.sync_copy(acc.at[pl.ds(r0, ZR)], ebuf.at[s])
        pltpu.async_copy(ebuf.at[s], out_hbm.at[cid, pl.ds(r0, ZR)],
                         csem.at[s])
    for z in range(NZ - NBUF, NZ):
        s = z % NBUF
        r0 = sid * RPT + z * ZR
        pltpu.make_async_copy(
            ebuf.at[s], out_hbm.at[cid, pl.ds(r0, ZR)], csem.at[s]).wait()


_sc_scatter = pl.kernel(
    _sc_body,
    out_type=jax.ShapeDtypeStruct((NC, N_PAD, H), jnp.float32),
    mesh=plsc.VectorSubcoreMesh(core_axis_name="c", subcore_axis_name="s"),
    scratch_types=[
        pltpu.VMEM_SHARED((N_PAD, H), jnp.float32),
        pltpu.VMEM((NBUF, CH, H), jnp.float32),
        pltpu.VMEM((TAIL, H), jnp.float32),
        pltpu.VMEM((NBUF, CH), jnp.int32),
        pltpu.VMEM((NBUF, CH), jnp.int32),
        pltpu.VMEM((TAIL,), jnp.int32),
        pltpu.VMEM((TAIL,), jnp.int32),
        pltpu.SemaphoreType.DMA((NBUF,)),
        pltpu.SemaphoreType.DMA((NBUF,)),
    ],
)


BR = 1000


def _mlp_body(x_ref, p_ref, w1xt_ref, w1at_ref, b1_ref,
              w2t_ref, b2_ref, o_ref):
    aggr = p_ref[0] + p_ref[1]
    h = jnp.dot(x_ref[...], w1xt_ref[...], preferred_element_type=jnp.float32)
    h = h + jnp.dot(aggr, w1at_ref[...], preferred_element_type=jnp.float32)
    h = jnp.maximum(h + b1_ref[...], 0.0)
    o_ref[...] = (
        jnp.dot(h, w2t_ref[...], preferred_element_type=jnp.float32)
        + b2_ref[...]
    )


_mlp = pl.pallas_call(
    _mlp_body,
    grid=(N_NODES // BR,),
    in_specs=[
        pl.BlockSpec((BR, H), lambda i: (i, 0)),
        pl.BlockSpec((NC, BR, H), lambda i: (0, i, 0)),
        pl.BlockSpec((H, H), lambda i: (0, 0)),
        pl.BlockSpec((H, H), lambda i: (0, 0)),
        pl.BlockSpec((1, H), lambda i: (0, 0)),
        pl.BlockSpec((H, H), lambda i: (0, 0)),
        pl.BlockSpec((1, H), lambda i: (0, 0)),
    ],
    out_specs=pl.BlockSpec((BR, H), lambda i: (i, 0)),
    out_shape=jax.ShapeDtypeStruct((N_NODES, H), jnp.float32),
)


def kernel(x, edge_index, edge_attr, W1, b1, W2, b2):
    row = edge_index[0].astype(jnp.int32)
    col = edge_index[1].astype(jnp.int32)
    zeros = jnp.zeros((ZR, H), jnp.float32)
    partials = _sc_scatter(zeros, row, col, edge_attr)
    return _mlp(x, partials,
                W1[:, :H].T, W1[:, H:].T, b1[None, :],
                W2.T, b2[None, :])
